# baseline (speedup 1.0000x reference)
import jax
import jax.numpy as jnp
import numpy as np
from jax.experimental import pallas as pl

_LOG2 = float(np.log(2.0))


def _mlp_pool_kernel(x_ref, mask_ref, w1_ref, b1_ref, w2_ref, b2_ref, out_ref):
    bb, a, d = x_ref.shape
    x = x_ref[...].reshape(bb * a, d)
    h = jnp.dot(x, w1_ref[...], preferred_element_type=jnp.float32)
    h = h + b1_ref[...]
    h = jax.nn.softplus(h) - _LOG2
    t = jnp.sum(h * w2_ref[...], axis=1).reshape(bb, a)
    t = (t + b2_ref[0, 0]) * mask_ref[...]
    out_ref[...] = jnp.sum(t, axis=1, keepdims=True)


def kernel(representation, atomic_numbers, atom_mask, W1, b1, W2, b2):
    B, A, D = representation.shape
    H = W1.shape[1]
    BB = 8

    b1r = b1.reshape(1, H)
    w2r = W2.reshape(1, H)
    b2r = b2.reshape(1, 1)

    y = pl.pallas_call(
        _mlp_pool_kernel,
        grid=(B // BB,),
        in_specs=[
            pl.BlockSpec((BB, A, D), lambda i: (i, 0, 0)),
            pl.BlockSpec((BB, A), lambda i: (i, 0)),
            pl.BlockSpec((D, H), lambda i: (0, 0)),
            pl.BlockSpec((1, H), lambda i: (0, 0)),
            pl.BlockSpec((1, H), lambda i: (0, 0)),
            pl.BlockSpec((1, 1), lambda i: (0, 0)),
        ],
        out_specs=pl.BlockSpec((BB, 1), lambda i: (i, 0)),
        out_shape=jax.ShapeDtypeStruct((B, 1), jnp.float32),
    )(representation, atom_mask, W1, b1r, w2r, b2r)
    return y


# speedup vs baseline: 1.6304x; 1.6304x over previous
import jax
import jax.numpy as jnp
import numpy as np
from jax.experimental import pallas as pl

_LOG2 = float(np.log(2.0))


def _mlp_pool_kernel(x_ref, mask_ref, w1_ref, b1_ref, w2_ref, b2_ref, out_ref):
    bb, a, d = x_ref.shape
    n = bb * a
    x = x_ref[...].reshape(n, d)
    h = jnp.dot(x, w1_ref[...], preferred_element_type=jnp.float32)
    h = jax.nn.softplus(h + b1_ref[...]) - _LOG2
    mask_tiled = jnp.concatenate([mask_ref[...]] * bb, axis=1)
    seg = jax.lax.broadcasted_iota(jnp.int32, (bb, n), 1) // a
    row = jax.lax.broadcasted_iota(jnp.int32, (bb, n), 0)
    mt = jnp.where(seg == row, mask_tiled, 0.0)
    q = jnp.dot(mt, h, preferred_element_type=jnp.float32)
    y = jnp.sum(q * w2_ref[...], axis=1, keepdims=True)
    msum = jnp.sum(mask_ref[...], axis=1, keepdims=True)
    out_ref[...] = y + b2_ref[0, 0] * msum


def kernel(representation, atomic_numbers, atom_mask, W1, b1, W2, b2):
    B, A, D = representation.shape
    H = W1.shape[1]
    BB = 8

    b1r = b1.reshape(1, H)
    w2r = W2.reshape(1, H)
    b2r = b2.reshape(1, 1)

    y = pl.pallas_call(
        _mlp_pool_kernel,
        grid=(B // BB,),
        in_specs=[
            pl.BlockSpec((BB, A, D), lambda i: (i, 0, 0)),
            pl.BlockSpec((BB, A), lambda i: (i, 0)),
            pl.BlockSpec((D, H), lambda i: (0, 0)),
            pl.BlockSpec((1, H), lambda i: (0, 0)),
            pl.BlockSpec((1, H), lambda i: (0, 0)),
            pl.BlockSpec((1, 1), lambda i: (0, 0)),
        ],
        out_specs=pl.BlockSpec((BB, 1), lambda i: (i, 0)),
        out_shape=jax.ShapeDtypeStruct((B, 1), jnp.float32),
    )(representation, atom_mask, W1, b1r, w2r, b2r)
    return y


# speedup vs baseline: 2.1384x; 1.3116x over previous
import jax
import jax.numpy as jnp
import numpy as np
from jax.experimental import pallas as pl

_LOG2 = float(np.log(2.0))
_LOG2E = float(np.log2(np.e))


def _mlp_pool_kernel(x_ref, mask_ref, w1_ref, b1_ref, w2_ref, c2_ref, out_ref):
    bb, a, d = x_ref.shape
    n = bb * a
    x = x_ref[...].reshape(n, d)
    t = jnp.dot(x, w1_ref[...], preferred_element_type=jnp.float32) + b1_ref[...]
    u = jnp.maximum(t, 0.0) + jnp.log2(1.0 + jnp.exp2(-jnp.abs(t)))
    mask_tiled = jnp.concatenate([mask_ref[...]] * bb, axis=1)
    seg = jax.lax.broadcasted_iota(jnp.int32, (bb, n), 1) // a
    row = jax.lax.broadcasted_iota(jnp.int32, (bb, n), 0)
    mt = jnp.where(seg == row, mask_tiled, 0.0)
    q = jnp.dot(mt, u, preferred_element_type=jnp.float32)
    y = jnp.sum(q * w2_ref[...], axis=1, keepdims=True)
    msum = jnp.sum(mask_ref[...], axis=1, keepdims=True)
    out_ref[...] = y + c2_ref[0, 0] * msum


def kernel(representation, atomic_numbers, atom_mask, W1, b1, W2, b2):
    B, A, D = representation.shape
    H = W1.shape[1]
    BB = 8

    w1s = W1 * _LOG2E
    b1s = (b1 * _LOG2E).reshape(1, H)
    w2l = (W2 * _LOG2).reshape(1, H)
    c2 = (b2 - _LOG2 * jnp.sum(W2)).reshape(1, 1)

    y = pl.pallas_call(
        _mlp_pool_kernel,
        grid=(B // BB,),
        in_specs=[
            pl.BlockSpec((BB, A, D), lambda i: (i, 0, 0)),
            pl.BlockSpec((BB, A), lambda i: (i, 0)),
            pl.BlockSpec((D, H), lambda i: (0, 0)),
            pl.BlockSpec((1, H), lambda i: (0, 0)),
            pl.BlockSpec((1, H), lambda i: (0, 0)),
            pl.BlockSpec((1, 1), lambda i: (0, 0)),
        ],
        out_specs=pl.BlockSpec((BB, 1), lambda i: (i, 0)),
        out_shape=jax.ShapeDtypeStruct((B, 1), jnp.float32),
    )(representation, atom_mask, w1s, b1s, w2l, c2)
    return y


# speedup vs baseline: 2.1488x; 1.0049x over previous
import jax
import jax.numpy as jnp
import numpy as np
from jax.experimental import pallas as pl

_LOG2 = float(np.log(2.0))
_LOG2E = float(np.log2(np.e))


def _mlp_pool_kernel(x_ref, mask_ref, w1_ref, b1_ref, w2_ref, c2_ref, out_ref):
    bb, a, d = x_ref.shape
    n = bb * a
    x = x_ref[...].reshape(n, d)
    t = jnp.dot(x, w1_ref[...], preferred_element_type=jnp.float32) + b1_ref[...]
    u = (jnp.maximum(t, 0.0) - 1.0) + jnp.log2(1.0 + jnp.exp2(-jnp.abs(t)))
    mask_tiled = jnp.concatenate([mask_ref[...]] * bb, axis=1)
    seg = jax.lax.broadcasted_iota(jnp.int32, (bb, n), 1) // a
    row = jax.lax.broadcasted_iota(jnp.int32, (bb, n), 0)
    mt = jnp.where(seg == row, mask_tiled, 0.0)
    q = jnp.dot(mt, u, preferred_element_type=jnp.float32)
    y = jnp.sum(q * w2_ref[...], axis=1, keepdims=True)
    msum = jnp.sum(mask_ref[...], axis=1, keepdims=True)
    out_ref[...] = y + c2_ref[0, 0] * msum


def kernel(representation, atomic_numbers, atom_mask, W1, b1, W2, b2):
    B, A, D = representation.shape
    H = W1.shape[1]
    BB = 8

    w1s = W1 * _LOG2E
    b1s = (b1 * _LOG2E).reshape(1, H)
    w2l = (W2 * _LOG2).reshape(1, H)
    c2 = b2.reshape(1, 1)

    y = pl.pallas_call(
        _mlp_pool_kernel,
        grid=(B // BB,),
        in_specs=[
            pl.BlockSpec((BB, A, D), lambda i: (i, 0, 0)),
            pl.BlockSpec((BB, A), lambda i: (i, 0)),
            pl.BlockSpec((D, H), lambda i: (0, 0)),
            pl.BlockSpec((1, H), lambda i: (0, 0)),
            pl.BlockSpec((1, H), lambda i: (0, 0)),
            pl.BlockSpec((1, 1), lambda i: (0, 0)),
        ],
        out_specs=pl.BlockSpec((BB, 1), lambda i: (i, 0)),
        out_shape=jax.ShapeDtypeStruct((B, 1), jnp.float32),
    )(representation, atom_mask, w1s, b1s, w2l, c2)
    return y


# speedup vs baseline: 2.2963x; 1.0686x over previous
import jax
import jax.numpy as jnp
import numpy as np
from jax.experimental import pallas as pl

_LOG2 = float(np.log(2.0))
_LOG2E = float(np.log2(np.e))


def _mlp_pool_kernel(x_ref, mask_ref, w1_ref, b1_ref, w2_ref, c2_ref, out_ref):
    bb, a, d = x_ref.shape
    n = bb * a
    x = x_ref[...].reshape(n, d)
    t = jnp.dot(x, w1_ref[...], preferred_element_type=jnp.float32) + b1_ref[...]
    u = (jnp.maximum(t, 0.0) - 1.0) + jnp.log2(1.0 + jnp.exp2(-jnp.abs(t)))
    mask_tiled = jnp.concatenate([mask_ref[...]] * bb, axis=1)
    seg = jax.lax.broadcasted_iota(jnp.int32, (bb, n), 1) // a
    row = jax.lax.broadcasted_iota(jnp.int32, (bb, n), 0)
    mt = jnp.where(seg == row, mask_tiled, 0.0)
    q = jnp.dot(mt, u, preferred_element_type=jnp.float32)
    y = jnp.sum(q * w2_ref[...], axis=1, keepdims=True)
    msum = jnp.sum(mask_ref[...], axis=1, keepdims=True)
    out_ref[...] = y + c2_ref[0, 0] * msum


def kernel(representation, atomic_numbers, atom_mask, W1, b1, W2, b2):
    B, A, D = representation.shape
    H = W1.shape[1]
    BB = 16

    w1s = W1 * _LOG2E
    b1s = (b1 * _LOG2E).reshape(1, H)
    w2l = (W2 * _LOG2).reshape(1, H)
    c2 = b2.reshape(1, 1)

    y = pl.pallas_call(
        _mlp_pool_kernel,
        grid=(B // BB,),
        in_specs=[
            pl.BlockSpec((BB, A, D), lambda i: (i, 0, 0)),
            pl.BlockSpec((BB, A), lambda i: (i, 0)),
            pl.BlockSpec((D, H), lambda i: (0, 0)),
            pl.BlockSpec((1, H), lambda i: (0, 0)),
            pl.BlockSpec((1, H), lambda i: (0, 0)),
            pl.BlockSpec((1, 1), lambda i: (0, 0)),
        ],
        out_specs=pl.BlockSpec((BB, 1), lambda i: (i, 0)),
        out_shape=jax.ShapeDtypeStruct((B, 1), jnp.float32),
    )(representation, atom_mask, w1s, b1s, w2l, c2)
    return y
